# Initial kernel scaffold; baseline (speedup 1.0000x reference)
import jax
import jax.numpy as jnp
from jax.experimental import pallas as pl


def kernel(x, batch, phi_w1, phi_b1, phi_w2, phi_b2, rho_w1, rho_b1, rho_w2, rho_b2):
    raise NotImplementedError("write your pallas kernel here")


# speedup vs baseline: 2.6459x; 2.6459x over previous
import jax
import jax.numpy as jnp
from jax.experimental import pallas as pl
from jax.experimental.pallas import tpu as pltpu

NSEG = 1000


def _pick_block(n):
    for b in (1000, 800, 500, 250, 200, 100, 50, 25, 20, 10, 8, 5, 4, 2, 1):
        if n % b == 0:
            return b
    return 1


def _k1(x_ref, brow_ref, w1_ref, b1_ref, w2_ref, b2_ref, rw1_ref,
        out1_ref, s_ref):
    i = pl.program_id(0)
    xb = x_ref[...]
    h = jnp.maximum(
        jnp.dot(xb, w1_ref[...], preferred_element_type=jnp.float32)
        + b1_ref[...], 0.0)
    out1_ref[...] = (
        jnp.dot(h, w2_ref[...], preferred_element_type=jnp.float32)
        + b2_ref[...])
    y = jnp.dot(xb, rw1_ref[...], preferred_element_type=jnp.float32)
    b = brow_ref[0]
    nb = b.shape[1]
    seg = jax.lax.broadcasted_iota(jnp.int32, (NSEG, nb), 0).astype(jnp.float32)
    oh_t = (jnp.broadcast_to(b, (NSEG, nb)) == seg).astype(jnp.float32)
    part = jnp.dot(oh_t, y, preferred_element_type=jnp.float32)

    @pl.when(i == 0)
    def _():
        s_ref[...] = part

    @pl.when(i > 0)
    def _():
        s_ref[...] += part


def _k2(out1_ref, bcol_ref, s_ref, rb1_ref, rw2_ref, rb2_ref, out_ref, r_scr):
    i = pl.program_id(0)

    @pl.when(i == 0)
    def _():
        r_scr[...] = (
            jnp.dot(jnp.maximum(s_ref[...] + rb1_ref[...], 0.0),
                    rw2_ref[...], preferred_element_type=jnp.float32)
            + rb2_ref[...])

    bc = bcol_ref[0]
    nb = bc.shape[0]
    seg = jax.lax.broadcasted_iota(jnp.int32, (nb, NSEG), 1).astype(jnp.float32)
    oh = (jnp.broadcast_to(bc, (nb, NSEG)) == seg).astype(jnp.float32)
    out_ref[...] = out1_ref[...] + jnp.dot(
        oh, r_scr[...], preferred_element_type=jnp.float32)


def kernel(x, batch, phi_w1, phi_b1, phi_w2, phi_b2,
           rho_w1, rho_b1, rho_w2, rho_b2):
    n, d_in = x.shape
    d_hid = phi_w1.shape[1]
    d_out = phi_w2.shape[1]
    bsz = _pick_block(n)
    nblk = n // bsz

    bf = batch.astype(jnp.float32)
    brow = bf.reshape(nblk, 1, bsz)
    bcol = bf.reshape(nblk, bsz, 1)
    b1 = phi_b1.reshape(1, d_hid)
    b2 = phi_b2.reshape(1, d_out)
    rb1 = rho_b1.reshape(1, d_hid)
    rb2 = rho_b2.reshape(1, d_out)

    full = lambda i: (0, 0)
    out1, s = pl.pallas_call(
        _k1,
        grid=(nblk,),
        in_specs=[
            pl.BlockSpec((bsz, d_in), lambda i: (i, 0)),
            pl.BlockSpec((1, 1, bsz), lambda i: (i, 0, 0)),
            pl.BlockSpec((d_in, d_hid), full),
            pl.BlockSpec((1, d_hid), full),
            pl.BlockSpec((d_hid, d_out), full),
            pl.BlockSpec((1, d_out), full),
            pl.BlockSpec((d_in, d_hid), full),
        ],
        out_specs=[
            pl.BlockSpec((bsz, d_out), lambda i: (i, 0)),
            pl.BlockSpec((NSEG, d_hid), full),
        ],
        out_shape=[
            jax.ShapeDtypeStruct((n, d_out), jnp.float32),
            jax.ShapeDtypeStruct((NSEG, d_hid), jnp.float32),
        ],
    )(x, brow, phi_w1, b1, phi_w2, b2, rho_w1)

    out = pl.pallas_call(
        _k2,
        grid=(nblk,),
        in_specs=[
            pl.BlockSpec((bsz, d_out), lambda i: (i, 0)),
            pl.BlockSpec((1, bsz, 1), lambda i: (i, 0, 0)),
            pl.BlockSpec((NSEG, d_hid), full),
            pl.BlockSpec((1, d_hid), full),
            pl.BlockSpec((d_hid, d_out), full),
            pl.BlockSpec((1, d_out), full),
        ],
        out_specs=pl.BlockSpec((bsz, d_out), lambda i: (i, 0)),
        out_shape=jax.ShapeDtypeStruct((n, d_out), jnp.float32),
        scratch_shapes=[pltpu.VMEM((NSEG, d_out), jnp.float32)],
    )(out1, bcol, s, rb1, rho_w2, rb2)
    return out
